# Initial kernel scaffold; baseline (speedup 1.0000x reference)
import jax
import jax.numpy as jnp
from jax import lax
from jax.experimental import pallas as pl
from jax.experimental.pallas import tpu as pltpu
from jax.experimental.pallas import tpu_sc as plsc

NUM_CORES = 2
NUM_SUBCORES = 16
NW = NUM_CORES * NUM_SUBCORES

CHUNK = 128
GROUP = 8


def _make_kernel(B, D):
    assert B % NW == 0
    per_w = B // NW
    assert per_w % CHUNK == 0
    chunks_per_w = per_w // CHUNK
    assert chunks_per_w % GROUP == 0
    n_groups = chunks_per_w // GROUP
    rows_per_group = GROUP * CHUNK

    mesh = plsc.VectorSubcoreMesh(core_axis_name="c", subcore_axis_name="s")

    def body(idx_hbm, table_hbm, out_hbm, idx_v, rows_v, sem):
        wid = lax.axis_index("s") * NUM_CORES + lax.axis_index("c")
        row0 = wid * chunks_per_w
        out0 = wid * per_w
        pltpu.sync_copy(idx_hbm.at[pl.ds(row0, chunks_per_w)], idx_v)

        def group(g, carry):
            handles = []
            for j in range(GROUP):
                h = pltpu.async_copy(
                    table_hbm.at[idx_v.at[g * GROUP + j]],
                    rows_v.at[pl.ds(j * CHUNK, CHUNK)],
                    sem,
                )
                handles.append(h)
            for h in handles:
                h.wait()
            pltpu.sync_copy(
                rows_v, out_hbm.at[pl.ds(out0 + g * rows_per_group, rows_per_group)]
            )
            return carry

        lax.fori_loop(0, n_groups, group, 0)

    return pl.kernel(
        body,
        out_type=jax.ShapeDtypeStruct((B, D), jnp.float32),
        mesh=mesh,
        scratch_types=[
            pltpu.VMEM((chunks_per_w, CHUNK), jnp.int32),
            pltpu.VMEM((rows_per_group, D), jnp.float32),
            pltpu.SemaphoreType.DMA,
        ],
    )


def kernel(token_ids, E):
    Bt, S = token_ids.shape
    V, D = E.shape
    B = Bt * S
    idx2d = token_ids.astype(jnp.int32).reshape(B // CHUNK, CHUNK)
    out = _make_kernel(B, D)(idx2d, E)
    return out.reshape(Bt, S, D)


# speedup vs baseline: 1.1030x; 1.1030x over previous
import jax
import jax.numpy as jnp
from jax import lax
from jax.experimental import pallas as pl
from jax.experimental.pallas import tpu as pltpu
from jax.experimental.pallas import tpu_sc as plsc

NUM_CORES = 2
NUM_SUBCORES = 16
NW = NUM_CORES * NUM_SUBCORES

CHUNK = 128
GROUP = 8


def _make_kernel(B, D):
    assert B % NW == 0
    per_w = B // NW
    assert per_w % CHUNK == 0
    chunks_per_w = per_w // CHUNK
    assert chunks_per_w % GROUP == 0
    n_groups = chunks_per_w // GROUP
    rows_per_group = GROUP * CHUNK

    mesh = plsc.VectorSubcoreMesh(core_axis_name="c", subcore_axis_name="s")

    def body(idx_hbm, table_hbm, out_hbm, idx_v, rows_v, sem):
        wid = lax.axis_index("s") * NUM_CORES + lax.axis_index("c")
        row0 = wid * chunks_per_w
        out0 = wid * per_w
        pltpu.sync_copy(idx_hbm.at[pl.ds(row0, chunks_per_w)], idx_v)

        def group(g, carry):
            handles = []
            for j in range(GROUP):
                h = pltpu.async_copy(
                    table_hbm.at[idx_v.at[g * GROUP + j]],
                    rows_v.at[pl.ds(j * CHUNK, CHUNK)],
                    sem,
                )
                handles.append(h)
            for h in handles:
                h.wait()
            pltpu.sync_copy(
                rows_v, out_hbm.at[pl.ds(out0 + g * rows_per_group, rows_per_group)]
            )
            return carry

        lax.fori_loop(0, n_groups, group, 0)

    return pl.kernel(
        body,
        out_type=jax.ShapeDtypeStruct((B, D), jnp.float32),
        mesh=mesh,
        scratch_types=[
            pltpu.VMEM((chunks_per_w, CHUNK), jnp.int32),
            pltpu.VMEM((rows_per_group, D), jnp.float32),
            pltpu.SemaphoreType.DMA,
        ],
        compiler_params=pltpu.CompilerParams(use_tc_tiling_on_sc=False),
    )


def kernel(token_ids, E):
    Bt, S = token_ids.shape
    V, D = E.shape
    B = Bt * S
    idx2d = token_ids.astype(jnp.int32).reshape(B // CHUNK, CHUNK)
    out = _make_kernel(B, D)(idx2d, E)
    return out.reshape(Bt, S, D)


# speedup vs baseline: 1.1039x; 1.0008x over previous
import jax
import jax.numpy as jnp
from jax import lax
from jax.experimental import pallas as pl
from jax.experimental.pallas import tpu as pltpu
from jax.experimental.pallas import tpu_sc as plsc

NUM_CORES = 2
NUM_SUBCORES = 16
NW = NUM_CORES * NUM_SUBCORES

CHUNK = 128
GROUP = 5
NBUF = 2


def _make_kernel(B, D):
    assert B % NW == 0
    per_w = B // NW
    assert per_w % CHUNK == 0
    chunks_per_w = per_w // CHUNK
    assert chunks_per_w % (GROUP * NBUF) == 0
    n_outer = chunks_per_w // (GROUP * NBUF)
    rows_per_group = GROUP * CHUNK

    mesh = plsc.VectorSubcoreMesh(core_axis_name="c", subcore_axis_name="s")

    def body(idx_hbm, table_hbm, out_hbm, idx_v, rows_v, gsem, wsem0, wsem1):
        wid = lax.axis_index("s") * NUM_CORES + lax.axis_index("c")
        row0 = wid * chunks_per_w
        out0 = wid * per_w
        wsems = [wsem0, wsem1]
        pltpu.sync_copy(idx_hbm.at[pl.ds(row0, chunks_per_w)], idx_v)

        def outer(t, carry):
            for b in range(NBUF):
                g = t * NBUF + b

                @pl.when(t > 0)
                def _():
                    pltpu.make_async_copy(
                        rows_v.at[b],
                        out_hbm.at[pl.ds(0, rows_per_group)],
                        wsems[b],
                    ).wait()

                handles = []
                for j in range(GROUP):
                    handles.append(pltpu.async_copy(
                        table_hbm.at[idx_v.at[g * GROUP + j]],
                        rows_v.at[b].at[pl.ds(j * CHUNK, CHUNK)],
                        gsem,
                    ))
                for h in handles:
                    h.wait()
                pltpu.async_copy(
                    rows_v.at[b],
                    out_hbm.at[pl.ds(out0 + g * rows_per_group, rows_per_group)],
                    wsems[b],
                )
            return carry

        lax.fori_loop(0, n_outer, outer, 0)
        for b in range(NBUF):
            pltpu.make_async_copy(
                rows_v.at[b], out_hbm.at[pl.ds(0, rows_per_group)], wsems[b]
            ).wait()

    return pl.kernel(
        body,
        out_type=jax.ShapeDtypeStruct((B, D), jnp.float32),
        mesh=mesh,
        scratch_types=[
            pltpu.VMEM((chunks_per_w, CHUNK), jnp.int32),
            pltpu.VMEM((NBUF, rows_per_group, D), jnp.float32),
            pltpu.SemaphoreType.DMA,
            pltpu.SemaphoreType.DMA,
            pltpu.SemaphoreType.DMA,
        ],
        compiler_params=pltpu.CompilerParams(use_tc_tiling_on_sc=False),
    )


def kernel(token_ids, E):
    Bt, S = token_ids.shape
    V, D = E.shape
    B = Bt * S
    idx2d = token_ids.astype(jnp.int32).reshape(B // CHUNK, CHUNK)
    out = _make_kernel(B, D)(idx2d, E)
    return out.reshape(Bt, S, D)
